# baseline (speedup 1.0000x reference)
import functools

import numpy as np
import jax
import jax.numpy as jnp
from jax import lax
from jax.experimental import pallas as pl
from jax.experimental.pallas import tpu as pltpu
from jax.experimental.pallas import tpu_sc as plsc

N = 10000
E = 320000
F = 128
NUM_RELS = 200
RPAD = 256
NB = 32
SI = 4
SO = 4

NC = 2
NS = 16
NW = NC * NS
CHB = 128
NCHUNK = E // CHB
NKMAX = 80
NKLAST = NCHUNK - (NW - 1) * NKMAX
PADC = NW * NKMAX
NPAD = 10240
RT = NPAD // NS

BE = 2560
GRID = E // BE

_PP = np.zeros((F, SI * F), np.float32)
for _i in range(SI):
    for _b in range(NB):
        for _j in range(SO):
            _PP[4 * _b + _i, 128 * _i + 4 * _b + _j] = 1.0


def _worker_chunks(wid):
    nk = jnp.where(wid < NW - 1, NKMAX, NKLAST)
    c0 = wid * NKMAX
    return nk, c0


def _sc_gather(x, src2):
    mesh = plsc.VectorSubcoreMesh(core_axis_name="c", subcore_axis_name="s")

    @functools.partial(
        pl.kernel,
        mesh=mesh,
        out_type=jax.ShapeDtypeStruct((E, F), jnp.float32),
        scratch_types=[
            pltpu.VMEM((NKMAX, CHB), jnp.int32),
            pltpu.VMEM((4, CHB, F), jnp.float32),
            [pltpu.SemaphoreType.DMA] * 4,
            [pltpu.SemaphoreType.DMA] * 4,
        ],
    )
    def k(x_hbm, src_hbm, out_hbm, idx2d, rows, gsem, osem):
        wid = lax.axis_index("s") * NC + lax.axis_index("c")
        nk, c0 = _worker_chunks(wid)

        pltpu.sync_copy(src_hbm.at[pl.ds(c0, NKMAX)], idx2d)

        def gstart(ck, b):
            pltpu.async_copy(x_hbm.at[idx2d.at[ck]], rows.at[b], gsem[b])

        def gwait(b):
            pltpu.make_async_copy(x_hbm.at[idx2d.at[0]], rows.at[b],
                                  gsem[b]).wait()

        def ostart(ck, b):
            pltpu.async_copy(rows.at[b],
                             out_hbm.at[pl.ds((c0 + ck) * CHB, CHB)],
                             osem[b])

        def owait(b):
            pltpu.make_async_copy(rows.at[b], out_hbm.at[pl.ds(0, CHB)],
                                  osem[b]).wait()

        for b in range(3):
            gstart(b, b)

        def body(k4, carry):
            for b in range(4):
                ck = k4 * 4 + b

                @pl.when(ck < nk)
                def _():
                    gwait(b)

                    @pl.when(ck + 3 < nk)
                    def _():
                        @pl.when(ck >= 1)
                        def _():
                            owait((b + 3) % 4)

                        gstart(ck + 3, (b + 3) % 4)

                    ostart(ck, b)

            return carry

        lax.fori_loop(0, NKMAX // 4, body, 0)
        for b in range(4):
            owait(b)

    return k(x, src2)


def _sc_scatter_add(m, dst2, zeros):
    mesh = plsc.VectorSubcoreMesh(core_axis_name="c", subcore_axis_name="s")

    @functools.partial(
        pl.kernel,
        mesh=mesh,
        out_type=jax.ShapeDtypeStruct((NC, NPAD, F), jnp.float32),
        scratch_types=[
            pltpu.VMEM((NKMAX, CHB), jnp.int32),
            pltpu.VMEM((2, CHB, F), jnp.float32),
            pltpu.VMEM_SHARED((NPAD, F), jnp.float32),
            [pltpu.SemaphoreType.DMA] * 2,
            [pltpu.SemaphoreType.DMA] * 2,
        ],
    )
    def k(m_hbm, dst_hbm, zeros_hbm, out_hbm, idx2d, rows, acc, rsem, ssem):
        cid = lax.axis_index("c")
        sid = lax.axis_index("s")
        wid = sid * NC + cid
        nk, c0 = _worker_chunks(wid)

        pltpu.sync_copy(zeros_hbm.at[pl.ds(sid * RT, RT)],
                        acc.at[pl.ds(sid * RT, RT)])
        pltpu.sync_copy(dst_hbm.at[pl.ds(c0, NKMAX)], idx2d)
        plsc.subcore_barrier()

        def rstart(ck, b):
            pltpu.async_copy(m_hbm.at[pl.ds((c0 + ck) * CHB, CHB)],
                             rows.at[b], rsem[b])

        def rwait(b):
            pltpu.make_async_copy(m_hbm.at[pl.ds(0, CHB)], rows.at[b],
                                  rsem[b]).wait()

        def sstart(ck, b):
            pltpu.async_copy(rows.at[b], acc.at[idx2d.at[ck]], ssem[b],
                             add=True)

        def swait(b):
            pltpu.make_async_copy(rows.at[b], acc.at[idx2d.at[0]],
                                  ssem[b]).wait()

        rstart(0, 0)

        def body(k2, carry):
            for b in range(2):
                ck = k2 * 2 + b

                @pl.when(ck < nk)
                def _():
                    rwait(b)

                    @pl.when(ck + 1 < nk)
                    def _():
                        @pl.when(ck >= 1)
                        def _():
                            swait(1 - b)

                        rstart(ck + 1, 1 - b)

                    sstart(ck, b)

            return carry

        lax.fori_loop(0, NKMAX // 2, body, 0)
        for b in range(2):
            swait(b)
        plsc.subcore_barrier()
        pltpu.sync_copy(acc.at[pl.ds(sid * RT, RT)],
                        out_hbm.at[cid, pl.ds(sid * RT, RT)])

    return k(m, dst2, zeros)


def _tc_edge_body(rel_ref, xs_ref, ea_ref, t_ref, pp_ref, fcw_ref, out_ref):
    relv = rel_ref[...]
    rr = lax.broadcasted_iota(jnp.int32, (BE, RPAD), 1)
    oh = (relv == rr).astype(jnp.bfloat16)
    g = jnp.dot(oh, t_ref[...],
                preferred_element_type=jnp.float32).astype(jnp.bfloat16)
    xp = jnp.dot(xs_ref[...].astype(jnp.bfloat16), pp_ref[...],
                 preferred_element_type=jnp.float32).astype(jnp.bfloat16)

    def f32(v):
        return v.astype(jnp.float32)

    msg = (f32(xp[:, 0:128]) * f32(g[:, 0:128])
           + f32(xp[:, 128:256]) * f32(g[:, 128:256])
           + f32(xp[:, 256:384]) * f32(g[:, 256:384])
           + f32(xp[:, 384:512]) * f32(g[:, 384:512]))
    fce = jnp.dot(ea_ref[...], fcw_ref[...],
                  preferred_element_type=jnp.float32)
    t = f32(g[:, 512:640]) * fce
    feat = jnp.where(t > 0, t, 0.2 * t)
    out_ref[...] = msg + feat


def _tc_edge(rel2, xs, edge_attr, tbl, pp, fcw_t):
    return pl.pallas_call(
        _tc_edge_body,
        grid=(GRID,),
        in_specs=[
            pl.BlockSpec((BE, 1), lambda i: (i, 0)),
            pl.BlockSpec((BE, F), lambda i: (i, 0)),
            pl.BlockSpec((BE, F), lambda i: (i, 0)),
            pl.BlockSpec((RPAD, SI * F + F), lambda i: (0, 0)),
            pl.BlockSpec((F, SI * F), lambda i: (0, 0)),
            pl.BlockSpec((F, F), lambda i: (0, 0)),
        ],
        out_specs=pl.BlockSpec((BE, F), lambda i: (i, 0)),
        out_shape=jax.ShapeDtypeStruct((E, F), jnp.float32),
    )(rel2, xs, edge_attr, tbl, pp, fcw_t)


def _tc_combine_body(p0_ref, p1_ref, norm_ref, out_ref):
    out_ref[...] = (p0_ref[...] + p1_ref[...]) * norm_ref[...]


def _tc_combine(p0, p1, norm):
    br = 1000
    return pl.pallas_call(
        _tc_combine_body,
        grid=(N // br,),
        in_specs=[
            pl.BlockSpec((br, F), lambda i: (i, 0)),
            pl.BlockSpec((br, F), lambda i: (i, 0)),
            pl.BlockSpec((br, 1), lambda i: (i, 0)),
        ],
        out_specs=pl.BlockSpec((br, F), lambda i: (i, 0)),
        out_shape=jax.ShapeDtypeStruct((N, F), jnp.float32),
    )(p0, p1, norm)


def _pad_chunks(v):
    return jnp.pad(v.reshape(NCHUNK, CHB), ((0, PADC - NCHUNK), (0, 0)))


def kernel(x, norm, edge_attr, weight, attn, fc_w, edge_index, rel):
    src2 = _pad_chunks(edge_index[0])
    dst2 = _pad_chunks(edge_index[1])

    w4 = weight.reshape(NUM_RELS, NB, SI, SO)
    tw = jnp.transpose(w4, (0, 2, 1, 3)).reshape(NUM_RELS, SI * F)
    tbl = jnp.concatenate([tw, attn.reshape(NUM_RELS, F)], axis=1)
    tbl = jnp.pad(tbl, ((0, RPAD - NUM_RELS), (0, 0))).astype(jnp.bfloat16)
    pp = jnp.asarray(_PP, dtype=jnp.bfloat16)
    fcw_t = fc_w.T.astype(jnp.bfloat16)
    rel2 = rel.reshape(E, 1)

    xs = _sc_gather(x, src2)
    m = _tc_edge(rel2, xs, edge_attr.astype(jnp.bfloat16), tbl, pp, fcw_t)
    hp = _sc_scatter_add(m, dst2, jnp.zeros((NPAD, F), jnp.float32))
    return _tc_combine(hp[0, :N], hp[1, :N], norm)


# speedup vs baseline: 1.1213x; 1.1213x over previous
import functools

import numpy as np
import jax
import jax.numpy as jnp
from jax import lax
from jax.experimental import pallas as pl
from jax.experimental.pallas import tpu as pltpu
from jax.experimental.pallas import tpu_sc as plsc

N = 10000
E = 320000
F = 128
NUM_RELS = 200
RPAD = 256
NB = 32
SI = 4
SO = 4

NC = 2
NS = 16
NW = NC * NS
CHB = 128
NCHUNK = E // CHB
NKMAX = 80
NKLAST = NCHUNK - (NW - 1) * NKMAX
PADC = NW * NKMAX
NPAD = 10240
RT = NPAD // NS

BE = 2560
GRID = E // BE

_PP = np.zeros((F, SI * F), np.float32)
for _i in range(SI):
    for _b in range(NB):
        for _j in range(SO):
            _PP[4 * _b + _i, 128 * _i + 4 * _b + _j] = 1.0


def _worker_chunks(wid):
    nk = jnp.where(wid < NW - 1, NKMAX, NKLAST)
    c0 = wid * NKMAX
    return nk, c0


def _sc_gather(x, src2):
    mesh = plsc.VectorSubcoreMesh(core_axis_name="c", subcore_axis_name="s")

    @functools.partial(
        pl.kernel,
        mesh=mesh,
        out_type=jax.ShapeDtypeStruct((E, F), jnp.float32),
        scratch_types=[
            pltpu.VMEM((NKMAX, CHB), jnp.int32),
            pltpu.VMEM((4, CHB, F), jnp.float32),
            [pltpu.SemaphoreType.DMA] * 4,
            [pltpu.SemaphoreType.DMA] * 4,
        ],
    )
    def k(x_hbm, src_hbm, out_hbm, idx2d, rows, gsem, osem):
        wid = lax.axis_index("s") * NC + lax.axis_index("c")
        nk, c0 = _worker_chunks(wid)

        pltpu.sync_copy(src_hbm.at[pl.ds(c0, NKMAX)], idx2d)

        def gstart(ck, b):
            pltpu.async_copy(x_hbm.at[idx2d.at[ck]], rows.at[b], gsem[b])

        def gwait(b):
            pltpu.make_async_copy(x_hbm.at[idx2d.at[0]], rows.at[b],
                                  gsem[b]).wait()

        def ostart(ck, b):
            pltpu.async_copy(rows.at[b],
                             out_hbm.at[pl.ds((c0 + ck) * CHB, CHB)],
                             osem[b])

        def owait(b):
            pltpu.make_async_copy(rows.at[b], out_hbm.at[pl.ds(0, CHB)],
                                  osem[b]).wait()

        for b in range(3):
            gstart(b, b)

        def body(k4, carry):
            for b in range(4):
                ck = k4 * 4 + b

                @pl.when(ck < nk)
                def _():
                    gwait(b)

                    @pl.when(ck + 3 < nk)
                    def _():
                        @pl.when(ck >= 1)
                        def _():
                            owait((b + 3) % 4)

                        gstart(ck + 3, (b + 3) % 4)

                    ostart(ck, b)

            return carry

        lax.fori_loop(0, NKMAX // 4, body, 0)
        for b in range(4):
            owait(b)

    return k(x, src2)


def _sc_scatter_add(m, dst2, zeros):
    mesh = plsc.VectorSubcoreMesh(core_axis_name="c", subcore_axis_name="s")

    @functools.partial(
        pl.kernel,
        mesh=mesh,
        out_type=jax.ShapeDtypeStruct((NC, NPAD, F), jnp.float32),
        scratch_types=[
            pltpu.VMEM((NKMAX, CHB), jnp.int32),
            pltpu.VMEM((2, CHB, F), jnp.float32),
            pltpu.VMEM_SHARED((NPAD, F), jnp.float32),
            [pltpu.SemaphoreType.DMA] * 2,
            [pltpu.SemaphoreType.DMA] * 2,
        ],
    )
    def k(m_hbm, dst_hbm, zeros_hbm, out_hbm, idx2d, rows, acc, rsem, ssem):
        cid = lax.axis_index("c")
        sid = lax.axis_index("s")
        wid = sid * NC + cid
        nk, c0 = _worker_chunks(wid)

        pltpu.sync_copy(zeros_hbm.at[pl.ds(sid * RT, RT)],
                        acc.at[pl.ds(sid * RT, RT)])
        pltpu.sync_copy(dst_hbm.at[pl.ds(c0, NKMAX)], idx2d)
        plsc.subcore_barrier()

        def rstart(ck, b):
            pltpu.async_copy(m_hbm.at[pl.ds((c0 + ck) * CHB, CHB)],
                             rows.at[b], rsem[b])

        def rwait(b):
            pltpu.make_async_copy(m_hbm.at[pl.ds(0, CHB)], rows.at[b],
                                  rsem[b]).wait()

        def sstart(ck, b):
            pltpu.async_copy(rows.at[b], acc.at[idx2d.at[ck]], ssem[b],
                             add=True)

        def swait(b):
            pltpu.make_async_copy(rows.at[b], acc.at[idx2d.at[0]],
                                  ssem[b]).wait()

        rstart(0, 0)

        def body(k2, carry):
            for b in range(2):
                ck = k2 * 2 + b

                @pl.when(ck < nk)
                def _():
                    rwait(b)

                    @pl.when(ck + 1 < nk)
                    def _():
                        @pl.when(ck >= 1)
                        def _():
                            swait(1 - b)

                        rstart(ck + 1, 1 - b)

                    sstart(ck, b)

            return carry

        lax.fori_loop(0, NKMAX // 2, body, 0)
        for b in range(2):
            swait(b)
        plsc.subcore_barrier()
        pltpu.sync_copy(acc.at[pl.ds(sid * RT, RT)],
                        out_hbm.at[cid, pl.ds(sid * RT, RT)])

    return k(m, dst2, zeros)


def _tc_edge_body(rel_ref, xs_ref, ea_ref, t_ref, pp_ref, fcw_ref, out_ref):
    relv = rel_ref[...]
    rr = lax.broadcasted_iota(jnp.int32, (BE, RPAD), 1)
    oh = (relv == rr).astype(jnp.bfloat16)
    g = jnp.dot(oh, t_ref[...], preferred_element_type=jnp.float32)
    xp = jnp.dot(xs_ref[...].astype(jnp.bfloat16), pp_ref[...],
                 preferred_element_type=jnp.float32)
    msg = (xp[:, 0:128] * g[:, 0:128]
           + xp[:, 128:256] * g[:, 128:256]
           + xp[:, 256:384] * g[:, 256:384]
           + xp[:, 384:512] * g[:, 384:512])
    fce = jnp.dot(ea_ref[...].astype(jnp.bfloat16), fcw_ref[...],
                  preferred_element_type=jnp.float32)
    t = g[:, 512:640] * fce
    feat = jnp.where(t > 0, t, 0.2 * t)
    out_ref[...] = msg + feat


def _tc_edge(rel2, xs, edge_attr, tbl, pp, fcw_t):
    return pl.pallas_call(
        _tc_edge_body,
        grid=(GRID,),
        in_specs=[
            pl.BlockSpec((BE, 1), lambda i: (i, 0)),
            pl.BlockSpec((BE, F), lambda i: (i, 0)),
            pl.BlockSpec((BE, F), lambda i: (i, 0)),
            pl.BlockSpec((RPAD, SI * F + F), lambda i: (0, 0)),
            pl.BlockSpec((F, SI * F), lambda i: (0, 0)),
            pl.BlockSpec((F, F), lambda i: (0, 0)),
        ],
        out_specs=pl.BlockSpec((BE, F), lambda i: (i, 0)),
        out_shape=jax.ShapeDtypeStruct((E, F), jnp.float32),
    )(rel2, xs, edge_attr, tbl, pp, fcw_t)


def _tc_combine_body(hp_ref, norm_ref, out_ref):
    out_ref[...] = (hp_ref[0] + hp_ref[1]) * norm_ref[...]


def _tc_combine(hp, norm):
    br = 1000
    return pl.pallas_call(
        _tc_combine_body,
        grid=(N // br,),
        in_specs=[
            pl.BlockSpec((NC, br, F), lambda i: (0, i, 0)),
            pl.BlockSpec((br, 1), lambda i: (i, 0)),
        ],
        out_specs=pl.BlockSpec((br, F), lambda i: (i, 0)),
        out_shape=jax.ShapeDtypeStruct((N, F), jnp.float32),
    )(hp, norm)


def _pad_chunks(v):
    return jnp.pad(v.reshape(NCHUNK, CHB), ((0, PADC - NCHUNK), (0, 0)))


def kernel(x, norm, edge_attr, weight, attn, fc_w, edge_index, rel):
    src2 = _pad_chunks(edge_index[0])
    dst2 = _pad_chunks(edge_index[1])

    w4 = weight.reshape(NUM_RELS, NB, SI, SO)
    tw = jnp.transpose(w4, (0, 2, 1, 3)).reshape(NUM_RELS, SI * F)
    tbl = jnp.concatenate([tw, attn.reshape(NUM_RELS, F)], axis=1)
    tbl = jnp.pad(tbl, ((0, RPAD - NUM_RELS), (0, 0))).astype(jnp.bfloat16)
    pp = jnp.asarray(_PP, dtype=jnp.bfloat16)
    fcw_t = fc_w.T.astype(jnp.bfloat16)
    rel2 = rel.reshape(E, 1)

    xs = _sc_gather(x, src2)
    m = _tc_edge(rel2, xs, edge_attr, tbl, pp, fcw_t)
    hp = _sc_scatter_add(m, dst2, jnp.zeros((NPAD, F), jnp.float32))
    return _tc_combine(hp, norm)


# speedup vs baseline: 1.1969x; 1.0674x over previous
import functools

import numpy as np
import jax
import jax.numpy as jnp
from jax import lax
from jax.experimental import pallas as pl
from jax.experimental.pallas import tpu as pltpu
from jax.experimental.pallas import tpu_sc as plsc

N = 10000
E = 320000
F = 128
NUM_RELS = 200
RPAD = 256
NB = 32
SI = 4
SO = 4

NC = 2
NS = 16
NW = NC * NS
CHB = 128
NCHUNK = E // CHB
PADC = 2560
NKM = 40
HALF0 = NW * NKM
NPAD = 10240
RT = NPAD // NS

BE = 2560

_PP = np.zeros((F, SI * F), np.float32)
for _i in range(SI):
    for _b in range(NB):
        for _j in range(SO):
            _PP[4 * _b + _i, 128 * _i + 4 * _b + _j] = 1.0


def _sc_gather(x, src2, hb, eh):
    mesh = plsc.VectorSubcoreMesh(core_axis_name="c", subcore_axis_name="s")

    @functools.partial(
        pl.kernel,
        mesh=mesh,
        out_type=jax.ShapeDtypeStruct((eh, F), jnp.float32),
        scratch_types=[
            pltpu.VMEM((NKM, CHB), jnp.int32),
            pltpu.VMEM((4, CHB, F), jnp.float32),
            [pltpu.SemaphoreType.DMA] * 4,
            [pltpu.SemaphoreType.DMA] * 4,
        ],
    )
    def k(x_hbm, src_hbm, out_hbm, idx2d, rows, gsem, osem):
        wid = lax.axis_index("s") * NC + lax.axis_index("c")
        c0 = hb + NKM * wid
        nk = jnp.clip(NCHUNK - c0, 0, NKM)
        lb = NKM * wid * CHB

        pltpu.sync_copy(src_hbm.at[pl.ds(c0, NKM)], idx2d)

        def gstart(ck, b):
            pltpu.async_copy(x_hbm.at[idx2d.at[ck]], rows.at[b], gsem[b])

        def gwait(b):
            pltpu.make_async_copy(x_hbm.at[idx2d.at[0]], rows.at[b],
                                  gsem[b]).wait()

        def ostart(ck, b):
            pltpu.async_copy(rows.at[b],
                             out_hbm.at[pl.ds(lb + ck * CHB, CHB)],
                             osem[b])

        def owait(b):
            pltpu.make_async_copy(rows.at[b], out_hbm.at[pl.ds(0, CHB)],
                                  osem[b]).wait()

        @pl.when(nk > 0)
        def _():
            for b in range(3):
                gstart(b, b)

            def body(k4, carry):
                for b in range(4):
                    ck = k4 * 4 + b

                    @pl.when(ck < nk)
                    def _():
                        gwait(b)

                        @pl.when(ck + 3 < nk)
                        def _():
                            @pl.when(ck >= 1)
                            def _():
                                owait((b + 3) % 4)

                            gstart(ck + 3, (b + 3) % 4)

                        ostart(ck, b)

                return carry

            lax.fori_loop(0, NKM // 4, body, 0)
            for b in range(4):
                owait(b)

    return k(x, src2)


def _sc_scatter_add(m, dst2, zeros, hb):
    mesh = plsc.VectorSubcoreMesh(core_axis_name="c", subcore_axis_name="s")

    @functools.partial(
        pl.kernel,
        mesh=mesh,
        out_type=jax.ShapeDtypeStruct((NC, NPAD, F), jnp.float32),
        scratch_types=[
            pltpu.VMEM((NKM, CHB), jnp.int32),
            pltpu.VMEM((2, CHB, F), jnp.float32),
            pltpu.VMEM_SHARED((NPAD, F), jnp.float32),
            [pltpu.SemaphoreType.DMA] * 2,
            [pltpu.SemaphoreType.DMA] * 2,
        ],
    )
    def k(m_hbm, dst_hbm, zeros_hbm, out_hbm, idx2d, rows, acc, rsem, ssem):
        cid = lax.axis_index("c")
        sid = lax.axis_index("s")
        wid = sid * NC + cid
        c0 = hb + NKM * wid
        nk = jnp.clip(NCHUNK - c0, 0, NKM)
        lb = NKM * wid * CHB

        pltpu.sync_copy(zeros_hbm.at[pl.ds(sid * RT, RT)],
                        acc.at[pl.ds(sid * RT, RT)])
        pltpu.sync_copy(dst_hbm.at[pl.ds(c0, NKM)], idx2d)
        plsc.subcore_barrier()

        def rstart(ck, b):
            pltpu.async_copy(m_hbm.at[pl.ds(lb + ck * CHB, CHB)],
                             rows.at[b], rsem[b])

        def rwait(b):
            pltpu.make_async_copy(m_hbm.at[pl.ds(0, CHB)], rows.at[b],
                                  rsem[b]).wait()

        def sstart(ck, b):
            pltpu.async_copy(rows.at[b], acc.at[idx2d.at[ck]], ssem[b],
                             add=True)

        def swait(b):
            pltpu.make_async_copy(rows.at[b], acc.at[idx2d.at[0]],
                                  ssem[b]).wait()

        @pl.when(nk > 0)
        def _():
            rstart(0, 0)

            def body(k2, carry):
                for b in range(2):
                    ck = k2 * 2 + b

                    @pl.when(ck < nk)
                    def _():
                        rwait(b)

                        @pl.when(ck + 1 < nk)
                        def _():
                            @pl.when(ck >= 1)
                            def _():
                                swait(1 - b)

                            rstart(ck + 1, 1 - b)

                        sstart(ck, b)

                return carry

            lax.fori_loop(0, NKM // 2, body, 0)
            for b in range(2):
                swait(b)

        plsc.subcore_barrier()
        pltpu.sync_copy(acc.at[pl.ds(sid * RT, RT)],
                        out_hbm.at[cid, pl.ds(sid * RT, RT)])

    return k(m, dst2, zeros)


def _tc_edge_body(rel_ref, xs_ref, ea_ref, t_ref, pp_ref, fcw_ref, out_ref):
    relv = rel_ref[...]
    rr = lax.broadcasted_iota(jnp.int32, (BE, RPAD), 1)
    oh = (relv == rr).astype(jnp.bfloat16)
    g = jnp.dot(oh, t_ref[...], preferred_element_type=jnp.float32)
    xp = jnp.dot(xs_ref[...].astype(jnp.bfloat16), pp_ref[...],
                 preferred_element_type=jnp.float32)
    msg = (xp[:, 0:128] * g[:, 0:128]
           + xp[:, 128:256] * g[:, 128:256]
           + xp[:, 256:384] * g[:, 256:384]
           + xp[:, 384:512] * g[:, 384:512])
    fce = jnp.dot(ea_ref[...].astype(jnp.bfloat16), fcw_ref[...],
                  preferred_element_type=jnp.float32)
    t = g[:, 512:640] * fce
    feat = jnp.where(t > 0, t, 0.2 * t)
    out_ref[...] = msg + feat


def _tc_edge(rel2, xs, edge_attr, tbl, pp, fcw_t, boff, eh):
    grid = eh // BE
    return pl.pallas_call(
        _tc_edge_body,
        grid=(grid,),
        in_specs=[
            pl.BlockSpec((BE, 1), lambda i: (i + boff, 0)),
            pl.BlockSpec((BE, F), lambda i: (i, 0)),
            pl.BlockSpec((BE, F), lambda i: (i + boff, 0)),
            pl.BlockSpec((RPAD, SI * F + F), lambda i: (0, 0)),
            pl.BlockSpec((F, SI * F), lambda i: (0, 0)),
            pl.BlockSpec((F, F), lambda i: (0, 0)),
        ],
        out_specs=pl.BlockSpec((BE, F), lambda i: (i, 0)),
        out_shape=jax.ShapeDtypeStruct((eh, F), jnp.float32),
    )(rel2, xs, edge_attr, tbl, pp, fcw_t)


def _tc_combine_body(hp0_ref, hp1_ref, norm_ref, out_ref):
    out_ref[...] = (hp0_ref[0] + hp0_ref[1]
                    + hp1_ref[0] + hp1_ref[1]) * norm_ref[...]


def _tc_combine(hp0, hp1, norm):
    br = 1000
    return pl.pallas_call(
        _tc_combine_body,
        grid=(N // br,),
        in_specs=[
            pl.BlockSpec((NC, br, F), lambda i: (0, i, 0)),
            pl.BlockSpec((NC, br, F), lambda i: (0, i, 0)),
            pl.BlockSpec((br, 1), lambda i: (i, 0)),
        ],
        out_specs=pl.BlockSpec((br, F), lambda i: (i, 0)),
        out_shape=jax.ShapeDtypeStruct((N, F), jnp.float32),
    )(hp0, hp1, norm)


def _pad_chunks(v):
    return jnp.pad(v.reshape(NCHUNK, CHB), ((0, PADC - NCHUNK), (0, 0)))


def kernel(x, norm, edge_attr, weight, attn, fc_w, edge_index, rel):
    src2 = _pad_chunks(edge_index[0])
    dst2 = _pad_chunks(edge_index[1])

    w4 = weight.reshape(NUM_RELS, NB, SI, SO)
    tw = jnp.transpose(w4, (0, 2, 1, 3)).reshape(NUM_RELS, SI * F)
    tbl = jnp.concatenate([tw, attn.reshape(NUM_RELS, F)], axis=1)
    tbl = jnp.pad(tbl, ((0, RPAD - NUM_RELS), (0, 0))).astype(jnp.bfloat16)
    pp = jnp.asarray(_PP, dtype=jnp.bfloat16)
    fcw_t = fc_w.T.astype(jnp.bfloat16)
    rel2 = rel.reshape(E, 1)
    zeros = jnp.zeros((NPAD, F), jnp.float32)

    e0 = HALF0 * CHB
    e1 = E - e0
    xs0 = _sc_gather(x, src2, 0, e0)
    xs1 = _sc_gather(x, src2, HALF0, e1)
    m0 = _tc_edge(rel2, xs0, edge_attr, tbl, pp, fcw_t, 0, e0)
    m1 = _tc_edge(rel2, xs1, edge_attr, tbl, pp, fcw_t, e0 // BE, e1)
    hp0 = _sc_scatter_add(m0, dst2, zeros, 0)
    hp1 = _sc_scatter_add(m1, dst2, zeros, HALF0)
    return _tc_combine(hp0, hp1, norm)
